# Initial kernel scaffold; baseline (speedup 1.0000x reference)
import jax
import jax.numpy as jnp
from jax.experimental import pallas as pl


def kernel(x, edge_index, W1, b1, W2, b2, W3, b3, Wc, bc):
    raise NotImplementedError("write your pallas kernel here")


# speedup vs baseline: 16.6151x; 16.6151x over previous
import functools

import jax
import jax.numpy as jnp
from jax import lax
from jax.experimental import pallas as pl
from jax.experimental.pallas import tpu as pltpu
from jax.experimental.pallas import tpu_sc as plsc

N_NODES = 10000
N_PAD = 10240
DUMMY = N_NODES
N_EDGES = 320000
NT = 32
CHUNK = 128
NCH = -(-N_EDGES // (NT * CHUNK))
E_PAD = NT * NCH * CHUNK
ZROWS = N_PAD // 16
MBLK = 128
GRID = N_PAD // MBLK

_mesh = plsc.VectorSubcoreMesh(core_axis_name="c", subcore_axis_name="s")


@functools.partial(
    pl.kernel,
    out_type=jax.ShapeDtypeStruct((2, N_PAD), jnp.float32),
    mesh=_mesh,
    scratch_types=[
        pltpu.VMEM((NCH, CHUNK), jnp.int32),
        pltpu.VMEM((CHUNK,), jnp.float32),
        pltpu.VMEM((ZROWS,), jnp.float32),
        pltpu.VMEM_SHARED((N_PAD,), jnp.float32),
    ],
    compiler_params=pltpu.CompilerParams(use_tc_tiling_on_sc=False),
)
def _deg_kernel(dst_hbm, out_hbm, idx_v, ones_v, zer_v, acc):
    c = lax.axis_index("c")
    s = lax.axis_index("s")
    w = s * 2 + c

    def fill_ones(i, carry):
        ones_v[pl.ds(i * 16, 16)] = jnp.full((16,), 1.0, jnp.float32)
        return carry

    lax.fori_loop(0, CHUNK // 16, fill_ones, 0)

    def fill_zeros(i, carry):
        zer_v[pl.ds(i * 16, 16)] = jnp.zeros((16,), jnp.float32)
        return carry

    lax.fori_loop(0, ZROWS // 16, fill_zeros, 0)

    pltpu.sync_copy(zer_v, acc.at[pl.ds(s * ZROWS, ZROWS)])
    pltpu.sync_copy(dst_hbm.at[w], idx_v)
    plsc.subcore_barrier()

    def body(j, carry):
        pltpu.sync_copy(ones_v, acc.at[idx_v.at[j]], add=True)
        return carry

    lax.fori_loop(0, NCH, body, 0)
    plsc.subcore_barrier()
    pltpu.sync_copy(acc.at[pl.ds(s * ZROWS, ZROWS)],
                    out_hbm.at[c].at[pl.ds(s * ZROWS, ZROWS)])


def _make_agg(h):
    @functools.partial(
        pl.kernel,
        out_type=jax.ShapeDtypeStruct((2, N_PAD, h), jnp.float32),
        mesh=_mesh,
        scratch_types=[
            pltpu.VMEM((NCH, CHUNK), jnp.int32),
            pltpu.VMEM((NCH, CHUNK), jnp.int32),
            pltpu.VMEM((CHUNK, h), jnp.float32),
            pltpu.VMEM((CHUNK, h), jnp.float32),
            pltpu.VMEM_SHARED((N_PAD, h), jnp.float32),
            pltpu.SemaphoreType.DMA,
        ],
        compiler_params=pltpu.CompilerParams(use_tc_tiling_on_sc=False),
    )
    def _agg(g_hbm, src_hbm, dst_hbm, out_hbm, si_v, di_v, rows_v, zb_v, acc,
             sem):
        c = lax.axis_index("c")
        s = lax.axis_index("s")
        w = s * 2 + c

        def fill_zeros(i, carry):
            for k in range(h // 16):
                zb_v[i, pl.ds(k * 16, 16)] = jnp.zeros((16,), jnp.float32)
            return carry

        lax.fori_loop(0, CHUNK, fill_zeros, 0)
        for q in range(ZROWS // CHUNK):
            pltpu.sync_copy(zb_v, acc.at[pl.ds(s * ZROWS + q * CHUNK, CHUNK)])
        pltpu.sync_copy(src_hbm.at[w], si_v)
        pltpu.sync_copy(dst_hbm.at[w], di_v)
        plsc.subcore_barrier()

        def body(j, carry):
            pltpu.async_copy(g_hbm.at[si_v.at[j]], rows_v, sem).wait()
            pltpu.sync_copy(rows_v, acc.at[di_v.at[j]], add=True)
            return carry

        lax.fori_loop(0, NCH, body, 0)
        plsc.subcore_barrier()
        pltpu.sync_copy(acc.at[pl.ds(s * ZROWS, ZROWS)],
                        out_hbm.at[c].at[pl.ds(s * ZROWS, ZROWS)])

    return _agg


_agg64 = _make_agg(64)
_agg32 = _make_agg(32)
_agg16 = _make_agg(16)


def _dinv(deg_ref):
    d = deg_ref[0] + deg_ref[1] + 1.0
    return lax.rsqrt(jnp.maximum(d, 1.0))


def _t_first(deg_ref, x_ref, w_ref, o_ref):
    o_ref[...] = _dinv(deg_ref) * jnp.dot(
        x_ref[...], w_ref[...], preferred_element_type=jnp.float32)


def _t_mid(deg_ref, p_ref, g_ref, b_ref, w_ref, o_ref):
    dinv = _dinv(deg_ref)
    agg = p_ref[0] + p_ref[1] + g_ref[...]
    hid = jnp.maximum(dinv * agg + b_ref[...], 0.0)
    o_ref[...] = dinv * jnp.dot(hid, w_ref[...],
                                preferred_element_type=jnp.float32)


def _t_last(deg_ref, p_ref, g_ref, b_ref, w_ref, bc_ref, o_ref):
    dinv = _dinv(deg_ref)
    agg = p_ref[0] + p_ref[1] + g_ref[...]
    hid = jnp.maximum(dinv * agg + b_ref[...], 0.0)
    o_ref[...] = jnp.dot(hid, w_ref[...],
                         preferred_element_type=jnp.float32) + bc_ref[...]


def _deg_spec():
    return pl.BlockSpec((2, MBLK, 1), lambda m: (0, m, 0))


def _row_spec(h):
    return pl.BlockSpec((MBLK, h), lambda m: (m, 0))


def _p_spec(h):
    return pl.BlockSpec((2, MBLK, h), lambda m: (0, m, 0))


def _full_spec(shape):
    return pl.BlockSpec(shape, lambda m: tuple(0 for _ in shape))


def _call_first(degr, xp, w1):
    f_in, h = w1.shape
    return pl.pallas_call(
        _t_first,
        grid=(GRID,),
        in_specs=[_deg_spec(), _row_spec(f_in), _full_spec((f_in, h))],
        out_specs=_row_spec(h),
        out_shape=jax.ShapeDtypeStruct((N_PAD, h), jnp.float32),
    )(degr, xp, w1)


def _call_mid(degr, p, g, b, w):
    hp, hn = w.shape
    return pl.pallas_call(
        _t_mid,
        grid=(GRID,),
        in_specs=[_deg_spec(), _p_spec(hp), _row_spec(hp),
                  _full_spec((1, hp)), _full_spec((hp, hn))],
        out_specs=_row_spec(hn),
        out_shape=jax.ShapeDtypeStruct((N_PAD, hn), jnp.float32),
    )(degr, p, g, b.reshape(1, hp), w)


def _call_last(degr, p, g, b, wc, bc):
    hp, nc = wc.shape
    return pl.pallas_call(
        _t_last,
        grid=(GRID,),
        in_specs=[_deg_spec(), _p_spec(hp), _row_spec(hp),
                  _full_spec((1, hp)), _full_spec((hp, nc)),
                  _full_spec((1, nc))],
        out_specs=_row_spec(nc),
        out_shape=jax.ShapeDtypeStruct((N_PAD, nc), jnp.float32),
    )(degr, p, g, b.reshape(1, hp), wc, bc.reshape(1, nc))


def kernel(x, edge_index, W1, b1, W2, b2, W3, b3, Wc, bc):
    n = x.shape[0]
    pad_e = E_PAD - N_EDGES
    src = jnp.concatenate(
        [edge_index[0], jnp.zeros((pad_e,), jnp.int32)]).reshape(NT, NCH, CHUNK)
    dst = jnp.concatenate(
        [edge_index[1], jnp.full((pad_e,), DUMMY, jnp.int32)]).reshape(NT, NCH, CHUNK)
    xp = jnp.pad(x, ((0, N_PAD - n), (0, 0)))

    deg = _deg_kernel(dst)
    degr = deg.reshape(2, N_PAD, 1)

    g1 = _call_first(degr, xp, W1)
    p1 = _agg64(g1, src, dst)
    g2 = _call_mid(degr, p1, g1, b1, W2)
    p2 = _agg32(g2, src, dst)
    g3 = _call_mid(degr, p2, g2, b2, W3)
    p3 = _agg16(g3, src, dst)
    out = _call_last(degr, p3, g3, b3, Wc, bc)
    return out[:n]


# speedup vs baseline: 23.3630x; 1.4061x over previous
import functools

import jax
import jax.numpy as jnp
from jax import lax
from jax.experimental import pallas as pl
from jax.experimental.pallas import tpu as pltpu
from jax.experimental.pallas import tpu_sc as plsc

N_NODES = 10000
N_PAD = 10240
DUMMY = N_NODES
N_EDGES = 320000
NT = 32
CHUNK = 128
NBUF = 4
NCH = 80
E_PAD = NT * NCH * CHUNK
ZROWS = N_PAD // 16
MBLK = 1024
GRID = N_PAD // MBLK

_mesh = plsc.VectorSubcoreMesh(core_axis_name="c", subcore_axis_name="s")
_sc_params = pltpu.CompilerParams(use_tc_tiling_on_sc=False)


@functools.partial(
    pl.kernel,
    out_type=jax.ShapeDtypeStruct((2, N_PAD), jnp.float32),
    mesh=_mesh,
    scratch_types=[
        pltpu.VMEM((NCH, CHUNK), jnp.int32),
        pltpu.VMEM((CHUNK,), jnp.float32),
        pltpu.VMEM((ZROWS,), jnp.float32),
        pltpu.VMEM_SHARED((N_PAD,), jnp.float32),
    ],
    compiler_params=_sc_params,
)
def _deg_kernel(dst_hbm, out_hbm, idx_v, ones_v, zer_v, acc):
    c = lax.axis_index("c")
    s = lax.axis_index("s")
    w = s * 2 + c

    def fill_ones(i, carry):
        ones_v[pl.ds(i * 16, 16)] = jnp.full((16,), 1.0, jnp.float32)
        return carry

    lax.fori_loop(0, CHUNK // 16, fill_ones, 0)

    def fill_zeros(i, carry):
        zer_v[pl.ds(i * 16, 16)] = jnp.zeros((16,), jnp.float32)
        return carry

    lax.fori_loop(0, ZROWS // 16, fill_zeros, 0)

    pltpu.sync_copy(zer_v, acc.at[pl.ds(s * ZROWS, ZROWS)])
    pltpu.sync_copy(dst_hbm.at[w], idx_v)
    plsc.subcore_barrier()

    def body(j, carry):
        pltpu.sync_copy(ones_v, acc.at[idx_v.at[j]], add=True)
        return carry

    lax.fori_loop(0, NCH, body, 0)
    plsc.subcore_barrier()
    pltpu.sync_copy(acc.at[pl.ds(s * ZROWS, ZROWS)],
                    out_hbm.at[c].at[pl.ds(s * ZROWS, ZROWS)])


def _make_agg(h):
    @functools.partial(
        pl.kernel,
        out_type=jax.ShapeDtypeStruct((2, N_PAD, h), jnp.float32),
        mesh=_mesh,
        scratch_types=[
            pltpu.VMEM((NCH, CHUNK), jnp.int32),
            pltpu.VMEM((NCH, CHUNK), jnp.int32),
            pltpu.VMEM((NBUF, CHUNK, h), jnp.float32),
            pltpu.VMEM((CHUNK, h), jnp.float32),
            pltpu.VMEM_SHARED((N_PAD, h), jnp.float32),
        ] + [pltpu.SemaphoreType.DMA] * (2 * NBUF),
        compiler_params=_sc_params,
    )
    def _agg(g_hbm, src_hbm, dst_hbm, out_hbm, si_v, di_v, rows_v, zb_v, acc,
             *sems):
        gsem = sems[:NBUF]
        ssem = sems[NBUF:]
        c = lax.axis_index("c")
        s = lax.axis_index("s")
        w = s * 2 + c

        def fill_zeros(i, carry):
            for k in range(h // 16):
                zb_v[i, pl.ds(k * 16, 16)] = jnp.zeros((16,), jnp.float32)
            return carry

        lax.fori_loop(0, CHUNK, fill_zeros, 0)
        for q in range(ZROWS // CHUNK):
            pltpu.sync_copy(zb_v, acc.at[pl.ds(s * ZROWS + q * CHUNK, CHUNK)])
        pltpu.sync_copy(src_hbm.at[w], si_v)
        pltpu.sync_copy(dst_hbm.at[w], di_v)
        plsc.subcore_barrier()

        def gather_start(j, b):
            pltpu.async_copy(g_hbm.at[si_v.at[j]], rows_v.at[b], gsem[b])

        def gather_wait(j, b):
            pltpu.make_async_copy(g_hbm.at[si_v.at[j]], rows_v.at[b],
                                  gsem[b]).wait()

        def scatter_start(j, b):
            pltpu.async_copy(rows_v.at[b], acc.at[di_v.at[j]], ssem[b],
                             add=True)

        def scatter_wait(j, b):
            pltpu.make_async_copy(rows_v.at[b], acc.at[di_v.at[j]],
                                  ssem[b]).wait()

        for b in range(NBUF):
            gather_start(b, b)

        def body(i, carry):
            for b in range(NBUF):
                j = i * NBUF + b
                gather_wait(j, b)
                scatter_start(j, b)
                scatter_wait(j, b)
                gather_start(j + NBUF, b)
            return carry

        lax.fori_loop(0, NCH // NBUF - 1, body, 0)
        for b in range(NBUF):
            j = NCH - NBUF + b
            gather_wait(j, b)
            scatter_start(j, b)
            scatter_wait(j, b)
        plsc.subcore_barrier()
        pltpu.sync_copy(acc.at[pl.ds(s * ZROWS, ZROWS)],
                        out_hbm.at[c].at[pl.ds(s * ZROWS, ZROWS)])

    return _agg


_agg64 = _make_agg(64)
_agg32 = _make_agg(32)
_agg16 = _make_agg(16)


def _dinv(deg_ref):
    d = deg_ref[0, :] + deg_ref[1, :] + 1.0
    return lax.rsqrt(jnp.maximum(d, 1.0)).reshape(MBLK, 1)


def _t_first(deg_ref, x_ref, w_ref, o_ref):
    o_ref[...] = _dinv(deg_ref) * jnp.dot(
        x_ref[...], w_ref[...], preferred_element_type=jnp.float32)


def _t_mid(deg_ref, p_ref, g_ref, b_ref, w_ref, o_ref):
    dinv = _dinv(deg_ref)
    agg = p_ref[0] + p_ref[1] + g_ref[...]
    hid = jnp.maximum(dinv * agg + b_ref[...], 0.0)
    o_ref[...] = dinv * jnp.dot(hid, w_ref[...],
                                preferred_element_type=jnp.float32)


def _t_last(deg_ref, p_ref, g_ref, b_ref, w_ref, bc_ref, o_ref):
    dinv = _dinv(deg_ref)
    agg = p_ref[0] + p_ref[1] + g_ref[...]
    hid = jnp.maximum(dinv * agg + b_ref[...], 0.0)
    o_ref[...] = jnp.dot(hid, w_ref[...],
                         preferred_element_type=jnp.float32) + bc_ref[...]


def _deg_spec():
    return pl.BlockSpec((2, MBLK), lambda m: (0, m))


def _row_spec(h):
    return pl.BlockSpec((MBLK, h), lambda m: (m, 0))


def _p_spec(h):
    return pl.BlockSpec((2, MBLK, h), lambda m: (0, m, 0))


def _full_spec(shape):
    return pl.BlockSpec(shape, lambda m: tuple(0 for _ in shape))


def _call_first(deg, xp, w1):
    f_in, h = w1.shape
    return pl.pallas_call(
        _t_first,
        grid=(GRID,),
        in_specs=[_deg_spec(), _row_spec(f_in), _full_spec((f_in, h))],
        out_specs=_row_spec(h),
        out_shape=jax.ShapeDtypeStruct((N_PAD, h), jnp.float32),
    )(deg, xp, w1)


def _call_mid(deg, p, g, b, w):
    hp, hn = w.shape
    return pl.pallas_call(
        _t_mid,
        grid=(GRID,),
        in_specs=[_deg_spec(), _p_spec(hp), _row_spec(hp),
                  _full_spec((1, hp)), _full_spec((hp, hn))],
        out_specs=_row_spec(hn),
        out_shape=jax.ShapeDtypeStruct((N_PAD, hn), jnp.float32),
    )(deg, p, g, b.reshape(1, hp), w)


def _call_last(deg, p, g, b, wc, bc):
    hp, nc = wc.shape
    return pl.pallas_call(
        _t_last,
        grid=(GRID,),
        in_specs=[_deg_spec(), _p_spec(hp), _row_spec(hp),
                  _full_spec((1, hp)), _full_spec((hp, nc)),
                  _full_spec((1, nc))],
        out_specs=_row_spec(nc),
        out_shape=jax.ShapeDtypeStruct((N_PAD, nc), jnp.float32),
    )(deg, p, g, b.reshape(1, hp), wc, bc.reshape(1, nc))


def kernel(x, edge_index, W1, b1, W2, b2, W3, b3, Wc, bc):
    n = x.shape[0]
    ei = jnp.pad(edge_index, ((0, 0), (0, E_PAD - N_EDGES)),
                 constant_values=DUMMY)
    src = ei[0].reshape(NT, NCH, CHUNK)
    dst = ei[1].reshape(NT, NCH, CHUNK)
    xp = jnp.pad(x, ((0, N_PAD - n), (0, 0)))

    deg = _deg_kernel(dst)

    g1 = _call_first(deg, xp, W1)
    p1 = _agg64(g1, src, dst)
    g2 = _call_mid(deg, p1, g1, b1, W2)
    p2 = _agg32(g2, src, dst)
    g3 = _call_mid(deg, p2, g2, b2, W3)
    p3 = _agg16(g3, src, dst)
    out = _call_last(deg, p3, g3, b3, Wc, bc)
    return out[:n]
